# baseline (speedup 1.0000x reference)
import jax
import jax.numpy as jnp
from jax.experimental import pallas as pl
from jax.experimental.pallas import tpu as pltpu


def _fused_kernel(x_ref, w_ref, adj_ref, out_ref, s_ref):
    i = pl.program_id(0)
    k = x_ref.shape[1]
    f_out = w_ref.shape[1]

    @pl.when(i == 0)
    def _fill():
        w = w_ref[...]
        for c in range(k):
            s_ref[:, c * f_out:(c + 1) * f_out] = jnp.dot(
                x_ref[:, c, :], w, preferred_element_type=jnp.float32)

    out_ref[...] = jnp.maximum(
        jnp.dot(adj_ref[...], s_ref[...], preferred_element_type=jnp.float32),
        0.0)


def kernel(input, adj, weight):
    n, k, f_in = input.shape
    f_out = weight.shape[1]
    bm = 200

    out2d = pl.pallas_call(
        _fused_kernel,
        grid=(n // bm,),
        in_specs=[
            pl.BlockSpec((n, k, f_in), lambda i: (0, 0, 0)),
            pl.BlockSpec((f_in, f_out), lambda i: (0, 0)),
            pl.BlockSpec((bm, n), lambda i: (i, 0)),
        ],
        out_specs=pl.BlockSpec((bm, k * f_out), lambda i: (i, 0)),
        out_shape=jax.ShapeDtypeStruct((n, k * f_out), jnp.float32),
        scratch_shapes=[pltpu.VMEM((n, k * f_out), jnp.float32)],
    )(input, weight, adj)
    return out2d.reshape(n, k, f_out)


# speedup vs baseline: 1.0039x; 1.0039x over previous
import jax
import jax.numpy as jnp
from jax.experimental import pallas as pl
from jax.experimental.pallas import tpu as pltpu


def _fused_kernel(x_ref, w_ref, adj_ref, out_ref, s_ref):
    i = pl.program_id(0)
    j = pl.program_id(1)
    nj = pl.num_programs(1)
    bm, bk = adj_ref.shape
    k = x_ref.shape[1]
    f_out = w_ref.shape[1]
    n_rows = pl.num_programs(0) * bm

    @pl.when(i == 0)
    def _fill():
        w = w_ref[...]
        row = jax.lax.broadcasted_iota(jnp.int32, (bk, f_out), 0) + j * bk
        valid = row < n_rows
        for c in range(k):
            s_val = jnp.dot(x_ref[:, c, :], w,
                            preferred_element_type=jnp.float32)
            s_ref[pl.ds(j * bk, bk), c * f_out:(c + 1) * f_out] = jnp.where(
                valid, s_val, 0.0)

    a = adj_ref[...]
    col = jax.lax.broadcasted_iota(jnp.int32, (bm, bk), 1) + j * bk
    a = jnp.where(col < n_rows, a, 0.0)

    partial = jnp.dot(a, s_ref[pl.ds(j * bk, bk), :],
                      preferred_element_type=jnp.float32)

    @pl.when(j == 0)
    def _init():
        out_ref[...] = partial

    @pl.when(j > 0)
    def _acc():
        out_ref[...] += partial

    @pl.when(j == nj - 1)
    def _relu():
        out_ref[...] = jnp.maximum(out_ref[...], 0.0)


def kernel(input, adj, weight):
    n, k, f_in = input.shape
    f_out = weight.shape[1]
    bm = 2000
    bk = 1280
    nj = -(-n // bk)

    out2d = pl.pallas_call(
        _fused_kernel,
        grid=(n // bm, nj),
        in_specs=[
            pl.BlockSpec((bk, k, f_in),
                         lambda i, j: (jnp.where(i == 0, j, 0), 0, 0)),
            pl.BlockSpec((f_in, f_out), lambda i, j: (0, 0)),
            pl.BlockSpec((bm, bk), lambda i, j: (i, j)),
        ],
        out_specs=pl.BlockSpec((bm, k * f_out), lambda i, j: (i, 0)),
        out_shape=jax.ShapeDtypeStruct((n, k * f_out), jnp.float32),
        scratch_shapes=[pltpu.VMEM((nj * bk, k * f_out), jnp.float32)],
    )(input, weight, adj)
    return out2d.reshape(n, k, f_out)


# speedup vs baseline: 1.0418x; 1.0378x over previous
import jax
import jax.numpy as jnp
from jax.experimental import pallas as pl
from jax.experimental.pallas import tpu as pltpu


def _fused_kernel(x_ref, w_ref, adj_ref, out_ref, s_ref):
    i = pl.program_id(0)
    j = pl.program_id(1)
    nj = pl.num_programs(1)
    bj = x_ref.shape[0]
    k = x_ref.shape[1]
    f_out = w_ref.shape[1]

    @pl.when(i == 0)
    def _fill():
        w = w_ref[...]
        for c in range(k):
            s_ref[pl.ds(j * bj, bj), c * f_out:(c + 1) * f_out] = jnp.dot(
                x_ref[:, c, :], w, preferred_element_type=jnp.float32)

    @pl.when(j == nj - 1)
    def _compute():
        out_ref[...] = jnp.maximum(
            jnp.dot(adj_ref[...], s_ref[...],
                    preferred_element_type=jnp.float32),
            0.0)


def kernel(input, adj, weight):
    n, k, f_in = input.shape
    f_out = weight.shape[1]
    bm = 400
    bj = 1000

    out2d = pl.pallas_call(
        _fused_kernel,
        grid=(n // bm, n // bj),
        in_specs=[
            pl.BlockSpec((bj, k, f_in),
                         lambda i, j: (jnp.where(i == 0, j, 0), 0, 0)),
            pl.BlockSpec((f_in, f_out), lambda i, j: (0, 0)),
            pl.BlockSpec((bm, n), lambda i, j: (i, 0)),
        ],
        out_specs=pl.BlockSpec((bm, k * f_out), lambda i, j: (i, 0)),
        out_shape=jax.ShapeDtypeStruct((n, k * f_out), jnp.float32),
        scratch_shapes=[pltpu.VMEM((n, k * f_out), jnp.float32)],
    )(input, weight, adj)
    return out2d.reshape(n, k, f_out)
